# baseline (speedup 1.0000x reference)
import functools

import jax
import jax.numpy as jnp
from jax import lax
from jax.experimental import pallas as pl
from jax.experimental.pallas import tpu as pltpu
from jax.experimental.pallas import tpu_sc as plsc

_D = 128
_M = 32
_N = 10000
_ITERS = 3

_TOT = _N * _M
_K = 2
_CH = 40


def _sc_gather(x, idx3):
    info = plsc.get_sparse_core_info()
    nw = info.num_cores * info.num_subcores
    n_ch = idx3.shape[1]
    rows = nw * n_ch * _CH

    mesh = plsc.VectorSubcoreMesh(core_axis_name="c", subcore_axis_name="s")

    @functools.partial(
        pl.kernel,
        mesh=mesh,
        out_type=jax.ShapeDtypeStruct((rows, _D), jnp.float32),
        scratch_types=[
            pltpu.VMEM((n_ch, _CH), jnp.int32),
            pltpu.VMEM((_CH, _D), jnp.float32),
            pltpu.SemaphoreType.DMA,
        ],
    )
    def k(x_hbm, idx_hbm, z_hbm, idx_v, rows_v, sem):
        wid = lax.axis_index("s") * info.num_cores + lax.axis_index("c")
        chunk0 = wid * n_ch
        pltpu.sync_copy(idx_hbm.at[wid], idx_v)

        def body(j, carry):
            pltpu.async_copy(x_hbm.at[idx_v.at[j]], rows_v, sem).wait()
            base = (chunk0 + j) * _CH
            pltpu.sync_copy(rows_v, z_hbm.at[pl.ds(base, _CH)])
            return carry

        lax.fori_loop(0, n_ch, body, 0)

    return k(x, idx3)


_B = 200


def _routing_body(z_ref, x_ref, o_ref):
    b = _B
    zf = z_ref[...]
    xb = x_ref[...]

    ones = jnp.ones((_D, _D), jnp.float32)

    s = jnp.dot(zf * zf, ones)
    zf = zf * lax.rsqrt(jnp.maximum(s, 1e-24))
    sx = jnp.dot(xb * xb, ones)
    xb = xb * lax.rsqrt(jnp.maximum(sx, 1e-24))

    z = zf.reshape(b, _M, _D)

    u = jnp.sum(z, axis=1) * (1.0 / _M) + xb

    for it in range(1, _ITERS):
        n2 = jnp.dot(u * u, ones)
        u = u * (n2 * lax.rsqrt(jnp.maximum(n2, 1e-24)) / (n2 + 1.0))

        t = z * u[:, None, :]
        d = jnp.dot(t.reshape(b * _M, _D), ones).reshape(b, _M, _D)
        e = jnp.exp(d)
        num = jnp.sum(e * z, axis=1)
        den = jnp.sum(e, axis=1)
        u = num / den + xb

    o_ref[...] = u


def _tc_routing(z, x):
    n = x.shape[0]
    return pl.pallas_call(
        _routing_body,
        grid=(n // _B,),
        in_specs=[
            pl.BlockSpec((_B * _M, _D), lambda i: (i, 0)),
            pl.BlockSpec((_B, _D), lambda i: (i, 0)),
        ],
        out_specs=pl.BlockSpec((_B, _D), lambda i: (i, 0)),
        out_shape=jax.ShapeDtypeStruct((n, _D), jnp.float32),
    )(z, x)


def kernel(x, x_nb):
    idx = jnp.where(x_nb == 0, _N - 1, x_nb - 1).astype(jnp.int32)
    ns = _N // _K
    outs = []
    for s in range(_K):
        idx3 = idx[s * ns:(s + 1) * ns].reshape(32, ns * _M // (32 * _CH), _CH)
        z = _sc_gather(x, idx3)
        outs.append(_tc_routing(z, x[s * ns:(s + 1) * ns]))
    return jnp.concatenate(outs, axis=0)


# speedup vs baseline: 1.5488x; 1.5488x over previous
import functools

import jax
import jax.numpy as jnp
from jax import lax
from jax.experimental import pallas as pl
from jax.experimental.pallas import tpu as pltpu
from jax.experimental.pallas import tpu_sc as plsc

_D = 128
_M = 32
_N = 10000
_ITERS = 3

_TOT = _N * _M
_K = 2
_CH = 40


_NB = 5


def _sc_gather(x, idx3):
    info = plsc.get_sparse_core_info()
    nw = info.num_cores * info.num_subcores
    n_ch = idx3.shape[1]
    rows = nw * n_ch * _CH
    assert n_ch % _NB == 0

    mesh = plsc.VectorSubcoreMesh(core_axis_name="c", subcore_axis_name="s")

    @functools.partial(
        pl.kernel,
        mesh=mesh,
        out_type=jax.ShapeDtypeStruct((rows, _D), jnp.float32),
        scratch_types=[
            pltpu.VMEM((n_ch, _CH), jnp.int32),
            pltpu.VMEM((_NB, _CH, _D), jnp.float32),
            pltpu.SemaphoreType.DMA((_NB,)),
            pltpu.SemaphoreType.DMA((_NB,)),
        ],
    )
    def k(x_hbm, idx_hbm, z_hbm, idx_v, rows_v, gs, ws):
        wid = lax.axis_index("s") * info.num_cores + lax.axis_index("c")
        chunk0 = wid * n_ch
        pltpu.sync_copy(idx_hbm.at[wid], idx_v)

        def gather(j, b):
            return pltpu.make_async_copy(
                x_hbm.at[idx_v.at[j]], rows_v.at[b], gs.at[b])

        def wback(j, b):
            return pltpu.make_async_copy(
                rows_v.at[b], z_hbm.at[pl.ds((chunk0 + j) * _CH, _CH)],
                ws.at[b])

        for b in range(_NB - 1):
            gather(b, b).start()

        def outer(t, carry):
            for b in range(_NB):
                j = t * _NB + b
                jn = j + _NB - 1
                bn = (b + _NB - 1) % _NB

                @pl.when(jn < n_ch)
                def _():
                    @pl.when(j > 0)
                    def _():
                        wback(j - 1, bn).wait()
                    gather(jn, bn).start()

                gather(j, b).wait()
                wback(j, b).start()
            return carry

        lax.fori_loop(0, n_ch // _NB, outer, 0)
        for b in range(_NB):
            wback(n_ch - _NB + b, b).wait()

    return k(x, idx3)


_B = 200


def _routing_body(z_ref, x_ref, o_ref):
    b = _B
    zf = z_ref[...]
    xb = x_ref[...]

    ones = jnp.ones((_D, _D), jnp.float32)

    s = jnp.dot(zf * zf, ones)
    zf = zf * lax.rsqrt(jnp.maximum(s, 1e-24))
    sx = jnp.dot(xb * xb, ones)
    xb = xb * lax.rsqrt(jnp.maximum(sx, 1e-24))

    z = zf.reshape(b, _M, _D)

    u = jnp.sum(z, axis=1) * (1.0 / _M) + xb

    for it in range(1, _ITERS):
        n2 = jnp.dot(u * u, ones)
        u = u * (n2 * lax.rsqrt(jnp.maximum(n2, 1e-24)) / (n2 + 1.0))

        t = z * u[:, None, :]
        d = jnp.dot(t.reshape(b * _M, _D), ones).reshape(b, _M, _D)
        e = jnp.exp(d)
        num = jnp.sum(e * z, axis=1)
        den = jnp.sum(e, axis=1)
        u = num / den + xb

    o_ref[...] = u


def _tc_routing(z, x):
    n = x.shape[0]
    return pl.pallas_call(
        _routing_body,
        grid=(n // _B,),
        in_specs=[
            pl.BlockSpec((_B * _M, _D), lambda i: (i, 0)),
            pl.BlockSpec((_B, _D), lambda i: (i, 0)),
        ],
        out_specs=pl.BlockSpec((_B, _D), lambda i: (i, 0)),
        out_shape=jax.ShapeDtypeStruct((n, _D), jnp.float32),
    )(z, x)


def kernel(x, x_nb):
    idx = jnp.where(x_nb == 0, _N - 1, x_nb - 1).astype(jnp.int32)
    ns = _N // _K
    outs = []
    for s in range(_K):
        idx3 = idx[s * ns:(s + 1) * ns].reshape(32, ns * _M // (32 * _CH), _CH)
        z = _sc_gather(x, idx3)
        outs.append(_tc_routing(z, x[s * ns:(s + 1) * ns]))
    return jnp.concatenate(outs, axis=0)


# speedup vs baseline: 1.6220x; 1.0473x over previous
import functools

import jax
import jax.numpy as jnp
from jax import lax
from jax.experimental import pallas as pl
from jax.experimental.pallas import tpu as pltpu
from jax.experimental.pallas import tpu_sc as plsc

_D = 128
_M = 32
_N = 10000
_ITERS = 3

_TOT = _N * _M
_K = 5
_CH = 80


_NB = 5


def _sc_gather(x, idx3):
    info = plsc.get_sparse_core_info()
    nw = info.num_cores * info.num_subcores
    n_ch = idx3.shape[1]
    rows = nw * n_ch * _CH
    assert n_ch % _NB == 0

    mesh = plsc.VectorSubcoreMesh(core_axis_name="c", subcore_axis_name="s")

    @functools.partial(
        pl.kernel,
        mesh=mesh,
        out_type=jax.ShapeDtypeStruct((rows, _D), jnp.float32),
        scratch_types=[
            pltpu.VMEM((n_ch, _CH), jnp.int32),
            pltpu.VMEM((_NB, _CH, _D), jnp.float32),
            pltpu.SemaphoreType.DMA((_NB,)),
            pltpu.SemaphoreType.DMA((_NB,)),
        ],
    )
    def k(x_hbm, idx_hbm, z_hbm, idx_v, rows_v, gs, ws):
        wid = lax.axis_index("s") * info.num_cores + lax.axis_index("c")
        chunk0 = wid * n_ch
        pltpu.sync_copy(idx_hbm.at[wid], idx_v)

        def gather(j, b):
            return pltpu.make_async_copy(
                x_hbm.at[idx_v.at[j]], rows_v.at[b], gs.at[b])

        def wback(j, b):
            return pltpu.make_async_copy(
                rows_v.at[b], z_hbm.at[pl.ds((chunk0 + j) * _CH, _CH)],
                ws.at[b])

        for b in range(_NB - 1):
            gather(b, b).start()

        def outer(t, carry):
            for b in range(_NB):
                j = t * _NB + b
                jn = j + _NB - 1
                bn = (b + _NB - 1) % _NB

                @pl.when(jn < n_ch)
                def _():
                    @pl.when(j > 0)
                    def _():
                        wback(j - 1, bn).wait()
                    gather(jn, bn).start()

                gather(j, b).wait()
                wback(j, b).start()
            return carry

        lax.fori_loop(0, n_ch // _NB, outer, 0)
        for b in range(_NB):
            wback(n_ch - _NB + b, b).wait()

    return k(x, idx3)


_B = 200


def _routing_body(z_ref, x_ref, o_ref):
    b = _B
    zf = z_ref[...]
    xb = x_ref[...]

    ones = jnp.ones((_D, _D), jnp.float32)

    s = jnp.dot(zf * zf, ones)
    zf = zf * lax.rsqrt(jnp.maximum(s, 1e-24))
    sx = jnp.dot(xb * xb, ones)
    xb = xb * lax.rsqrt(jnp.maximum(sx, 1e-24))

    z = zf.reshape(b, _M, _D)

    u = jnp.sum(z, axis=1) * (1.0 / _M) + xb

    for it in range(1, _ITERS):
        n2 = jnp.dot(u * u, ones)
        u = u * (n2 * lax.rsqrt(jnp.maximum(n2, 1e-24)) / (n2 + 1.0))

        t = z * u[:, None, :]
        d = jnp.dot(t.reshape(b * _M, _D), ones).reshape(b, _M, _D)
        e = jnp.exp(d)
        num = jnp.sum(e * z, axis=1)
        den = jnp.sum(e, axis=1)
        u = num / den + xb

    o_ref[...] = u


def _tc_routing(z, x):
    n = x.shape[0]
    return pl.pallas_call(
        _routing_body,
        grid=(n // _B,),
        in_specs=[
            pl.BlockSpec((_B * _M, _D), lambda i: (i, 0)),
            pl.BlockSpec((_B, _D), lambda i: (i, 0)),
        ],
        out_specs=pl.BlockSpec((_B, _D), lambda i: (i, 0)),
        out_shape=jax.ShapeDtypeStruct((n, _D), jnp.float32),
    )(z, x)


def kernel(x, x_nb):
    idx = jnp.where(x_nb == 0, _N - 1, x_nb - 1).astype(jnp.int32)
    ns = _N // _K
    outs = []
    for s in range(_K):
        idx3 = idx[s * ns:(s + 1) * ns].reshape(32, ns * _M // (32 * _CH), _CH)
        z = _sc_gather(x, idx3)
        outs.append(_tc_routing(z, x[s * ns:(s + 1) * ns]))
    return jnp.concatenate(outs, axis=0)


# speedup vs baseline: 1.6221x; 1.0000x over previous
import functools

import jax
import jax.numpy as jnp
from jax import lax
from jax.experimental import pallas as pl
from jax.experimental.pallas import tpu as pltpu
from jax.experimental.pallas import tpu_sc as plsc

_D = 128
_M = 32
_N = 10000
_ITERS = 3

_TOT = _N * _M
_K = 5
_CH = 80


_NB = 5


def _sc_gather(x, idx3):
    info = plsc.get_sparse_core_info()
    nw = info.num_cores * info.num_subcores
    n_ch = idx3.shape[1]
    rows = nw * n_ch * _CH
    assert n_ch % _NB == 0

    mesh = plsc.VectorSubcoreMesh(core_axis_name="c", subcore_axis_name="s")

    @functools.partial(
        pl.kernel,
        mesh=mesh,
        out_type=jax.ShapeDtypeStruct((rows, _D), x.dtype),
        scratch_types=[
            pltpu.VMEM((n_ch, _CH), jnp.int32),
            pltpu.VMEM((_NB, _CH, _D), x.dtype),
            pltpu.SemaphoreType.DMA((_NB,)),
            pltpu.SemaphoreType.DMA((_NB,)),
        ],
    )
    def k(x_hbm, idx_hbm, z_hbm, idx_v, rows_v, gs, ws):
        wid = lax.axis_index("s") * info.num_cores + lax.axis_index("c")
        chunk0 = wid * n_ch
        pltpu.sync_copy(idx_hbm.at[wid], idx_v)

        def gather(j, b):
            return pltpu.make_async_copy(
                x_hbm.at[idx_v.at[j]], rows_v.at[b], gs.at[b])

        def wback(j, b):
            return pltpu.make_async_copy(
                rows_v.at[b], z_hbm.at[pl.ds((chunk0 + j) * _CH, _CH)],
                ws.at[b])

        for b in range(_NB - 1):
            gather(b, b).start()

        def outer(t, carry):
            for b in range(_NB):
                j = t * _NB + b
                jn = j + _NB - 1
                bn = (b + _NB - 1) % _NB

                @pl.when(jn < n_ch)
                def _():
                    @pl.when(j > 0)
                    def _():
                        wback(j - 1, bn).wait()
                    gather(jn, bn).start()

                gather(j, b).wait()
                wback(j, b).start()
            return carry

        lax.fori_loop(0, n_ch // _NB, outer, 0)
        for b in range(_NB):
            wback(n_ch - _NB + b, b).wait()

    return k(x, idx3)


_B = 200


def _routing_body(z_ref, x_ref, o_ref):
    b = _B
    zf = z_ref[...].astype(jnp.float32)
    xb = x_ref[...]

    ones = jnp.ones((_D, _D), jnp.float32)

    s = jnp.dot(zf * zf, ones)
    zf = zf * lax.rsqrt(jnp.maximum(s, 1e-24))
    sx = jnp.dot(xb * xb, ones)
    xb = xb * lax.rsqrt(jnp.maximum(sx, 1e-24))

    z = zf.reshape(b, _M, _D)

    u = jnp.sum(z, axis=1) * (1.0 / _M) + xb

    for it in range(1, _ITERS):
        n2 = jnp.dot(u * u, ones)
        u = u * (n2 * lax.rsqrt(jnp.maximum(n2, 1e-24)) / (n2 + 1.0))

        t = z * u[:, None, :]
        d = jnp.dot(t.reshape(b * _M, _D), ones).reshape(b, _M, _D)
        e = jnp.exp(d)
        num = jnp.sum(e * z, axis=1)
        den = jnp.sum(e, axis=1)
        u = num / den + xb

    o_ref[...] = u


def _tc_routing(z, x):
    n = x.shape[0]
    return pl.pallas_call(
        _routing_body,
        grid=(n // _B,),
        in_specs=[
            pl.BlockSpec((_B * _M, _D), lambda i: (i, 0)),
            pl.BlockSpec((_B, _D), lambda i: (i, 0)),
        ],
        out_specs=pl.BlockSpec((_B, _D), lambda i: (i, 0)),
        out_shape=jax.ShapeDtypeStruct((n, _D), jnp.float32),
    )(z, x)


def kernel(x, x_nb):
    idx = jnp.where(x_nb == 0, _N - 1, x_nb - 1).astype(jnp.int32)
    ns = _N // _K
    outs = []
    for s in range(_K):
        idx3 = idx[s * ns:(s + 1) * ns].reshape(32, ns * _M // (32 * _CH), _CH)
        z = _sc_gather(x, idx3)
        outs.append(_tc_routing(z, x[s * ns:(s + 1) * ns]))
    return jnp.concatenate(outs, axis=0)


# speedup vs baseline: 1.6501x; 1.0173x over previous
import functools

import jax
import jax.numpy as jnp
from jax import lax
from jax.experimental import pallas as pl
from jax.experimental.pallas import tpu as pltpu
from jax.experimental.pallas import tpu_sc as plsc

_D = 128
_M = 32
_N = 10000
_ITERS = 3

_TOT = _N * _M
_K = 5
_CH = 80


_NB = 5


def _sc_gather(x, idx3):
    info = plsc.get_sparse_core_info()
    nw = info.num_cores * info.num_subcores
    n_ch = idx3.shape[1]
    rows = nw * n_ch * _CH
    assert n_ch % _NB == 0

    mesh = plsc.VectorSubcoreMesh(core_axis_name="c", subcore_axis_name="s")

    @functools.partial(
        pl.kernel,
        mesh=mesh,
        out_type=jax.ShapeDtypeStruct((rows, _D), x.dtype),
        scratch_types=[
            pltpu.VMEM((n_ch, _CH), jnp.int32),
            pltpu.VMEM((_NB, _CH, _D), x.dtype),
            pltpu.SemaphoreType.DMA((_NB,)),
            pltpu.SemaphoreType.DMA((_NB,)),
        ],
    )
    def k(x_hbm, idx_hbm, z_hbm, idx_v, rows_v, gs, ws):
        wid = lax.axis_index("s") * info.num_cores + lax.axis_index("c")
        chunk0 = wid * n_ch
        pltpu.sync_copy(idx_hbm.at[wid], idx_v)

        def gather(j, b):
            return pltpu.make_async_copy(
                x_hbm.at[idx_v.at[j]], rows_v.at[b], gs.at[b])

        def wback(j, b):
            return pltpu.make_async_copy(
                rows_v.at[b], z_hbm.at[pl.ds((chunk0 + j) * _CH, _CH)],
                ws.at[b])

        for b in range(_NB - 1):
            gather(b, b).start()

        def outer(t, carry):
            for b in range(_NB):
                j = t * _NB + b
                jn = j + _NB - 1
                bn = (b + _NB - 1) % _NB

                @pl.when(jn < n_ch)
                def _():
                    @pl.when(j > 0)
                    def _():
                        wback(j - 1, bn).wait()
                    gather(jn, bn).start()

                gather(j, b).wait()
                wback(j, b).start()
            return carry

        lax.fori_loop(0, n_ch // _NB, outer, 0)
        for b in range(_NB):
            wback(n_ch - _NB + b, b).wait()

    return k(x, idx3)


_B = 400


def _routing_body(z_ref, x_ref, o_ref):
    b = _B
    zf = z_ref[...].astype(jnp.float32)
    xb = x_ref[...]

    ones = jnp.ones((_D, _D), jnp.float32)

    s = jnp.dot(zf * zf, ones)
    zf = zf * lax.rsqrt(jnp.maximum(s, 1e-24))
    sx = jnp.dot(xb * xb, ones)
    xb = xb * lax.rsqrt(jnp.maximum(sx, 1e-24))

    z = zf.reshape(b, _M, _D)

    u = jnp.sum(z, axis=1) * (1.0 / _M) + xb

    for it in range(1, _ITERS):
        n2 = jnp.dot(u * u, ones)
        u = u * (n2 * lax.rsqrt(jnp.maximum(n2, 1e-24)) / (n2 + 1.0))

        t = z * u[:, None, :]
        d = jnp.dot(t.reshape(b * _M, _D), ones).reshape(b, _M, _D)
        e = jnp.exp(d)
        num = jnp.sum(e * z, axis=1)
        den = jnp.sum(e, axis=1)
        u = num / den + xb

    o_ref[...] = u


def _tc_routing(z, x):
    n = x.shape[0]
    return pl.pallas_call(
        _routing_body,
        grid=(n // _B,),
        in_specs=[
            pl.BlockSpec((_B * _M, _D), lambda i: (i, 0)),
            pl.BlockSpec((_B, _D), lambda i: (i, 0)),
        ],
        out_specs=pl.BlockSpec((_B, _D), lambda i: (i, 0)),
        out_shape=jax.ShapeDtypeStruct((n, _D), jnp.float32),
    )(z, x)


def kernel(x, x_nb):
    idx = jnp.where(x_nb == 0, _N - 1, x_nb - 1).astype(jnp.int32)
    ns = _N // _K
    outs = []
    for s in range(_K):
        idx3 = idx[s * ns:(s + 1) * ns].reshape(32, ns * _M // (32 * _CH), _CH)
        z = _sc_gather(x, idx3)
        outs.append(_tc_routing(z, x[s * ns:(s + 1) * ns]))
    return jnp.concatenate(outs, axis=0)
